# baseline (speedup 1.0000x reference)
import functools

import jax
import jax.numpy as jnp
from jax import lax
from jax.experimental import pallas as pl
from jax.experimental.pallas import tpu as pltpu
from jax.experimental.pallas import tpu_sc as plsc


def _chunk_offsets(total):
    sizes = []
    left = total
    while left > 128:
        sizes.append(128)
        left -= 128
    if left > 64:
        sizes.append(64)
        left -= 64
    sizes.append(left)
    offs, o = [], 0
    for s in sizes:
        offs.append((o, s))
        o += s
    return tuple(offs)


@functools.lru_cache(maxsize=None)
def _make_gather(vocab, dim, batch):
    info = plsc.get_sparse_core_info()
    num_sub = info.num_subcores
    num_workers = info.num_cores * num_sub
    b_per_w = batch // num_workers
    assert batch % num_workers == 0
    chunks = _chunk_offsets(b_per_w)
    n_chunks = len(chunks)
    n_hbm = min(2, n_chunks - 1)
    v_share = (-(-vocab // num_sub) + 7) // 8 * 8
    n_copiers = vocab // v_share
    v_rem = vocab - n_copiers * v_share
    assert v_rem % 8 == 0 and n_copiers + (1 if v_rem else 0) <= num_sub

    mesh = plsc.VectorSubcoreMesh(core_axis_name="c", subcore_axis_name="s")

    @functools.partial(
        pl.kernel,
        mesh=mesh,
        out_type=jax.ShapeDtypeStruct((batch, dim), jnp.float32),
        scratch_types=[
            pltpu.VMEM((b_per_w,), jnp.int32),
            pltpu.VMEM((b_per_w, dim), jnp.float32),
            pltpu.VMEM_SHARED((vocab, dim), jnp.float32),
            pltpu.SemaphoreType.DMA((n_chunks,)),
            pltpu.SemaphoreType.DMA,
            pltpu.SemaphoreType.DMA,
        ],
    )
    def gather_kernel(idx_hbm, table_hbm, out_hbm, idx_v, rows_v, table_sp,
                      gsem, wsem, ssem):
        sid = lax.axis_index("s")
        wid = sid * info.num_cores + lax.axis_index("c")
        base = wid * b_per_w
        if v_rem:
            @pl.when(sid == n_copiers)
            def _copy_rem():
                pltpu.async_copy(
                    table_hbm.at[pl.ds(n_copiers * v_share, v_rem)],
                    table_sp.at[pl.ds(n_copiers * v_share, v_rem)],
                    ssem)

        @pl.when(sid < n_copiers)
        def _copy_share():
            row0 = sid * v_share
            pltpu.async_copy(table_hbm.at[pl.ds(row0, v_share)],
                             table_sp.at[pl.ds(row0, v_share)],
                             ssem)

        pltpu.sync_copy(idx_hbm.at[pl.ds(base, b_per_w)], idx_v)

        gathers = [
            pltpu.async_copy(
                table_hbm.at[idx_v.at[pl.ds(o, s)]],
                rows_v.at[pl.ds(o, s)],
                gsem.at[j],
            )
            for j, (o, s) in enumerate(chunks)
            if j < n_hbm
        ]

        if v_rem:
            @pl.when(sid == n_copiers)
            def _wait_rem():
                pltpu.make_async_copy(
                    table_hbm.at[pl.ds(n_copiers * v_share, v_rem)],
                    table_sp.at[pl.ds(n_copiers * v_share, v_rem)],
                    ssem).wait()

        @pl.when(sid < n_copiers)
        def _wait_share():
            row0 = sid * v_share
            pltpu.make_async_copy(table_hbm.at[pl.ds(row0, v_share)],
                                  table_sp.at[pl.ds(row0, v_share)],
                                  ssem).wait()

        plsc.subcore_barrier()
        gathers += [
            pltpu.async_copy(
                table_sp.at[idx_v.at[pl.ds(o, s)]],
                rows_v.at[pl.ds(o, s)],
                gsem.at[j],
            )
            for j, (o, s) in enumerate(chunks)
            if j >= n_hbm
        ]
        writes = []
        for j, (o, s) in enumerate(chunks):
            gathers[j].wait()
            writes.append(
                pltpu.async_copy(
                    rows_v.at[pl.ds(o, s)],
                    out_hbm.at[pl.ds(base + o, s)],
                    wsem,
                )
            )
        for w in writes:
            w.wait()

    return gather_kernel


def kernel(timesteps, table):
    batch = timesteps.shape[0]
    vocab, dim = table.shape
    return _make_gather(vocab, dim, batch)(timesteps.astype(jnp.int32), table)


# speedup vs baseline: 1.0827x; 1.0827x over previous
import functools

import jax
import jax.numpy as jnp
from jax import lax
from jax.experimental import pallas as pl
from jax.experimental.pallas import tpu as pltpu
from jax.experimental.pallas import tpu_sc as plsc


def _chunk_offsets(total):
    sizes = []
    left = total
    while left > 128:
        sizes.append(128)
        left -= 128
    if left > 64:
        sizes.append(64)
        left -= 64
    sizes.append(left)
    offs, o = [], 0
    for s in sizes:
        offs.append((o, s))
        o += s
    return tuple(offs)


@functools.lru_cache(maxsize=None)
def _make_gather(vocab, dim, batch):
    info = plsc.get_sparse_core_info()
    num_sub = info.num_subcores
    num_workers = info.num_cores * num_sub
    b_per_w = batch // num_workers
    assert batch % num_workers == 0
    chunks = _chunk_offsets(b_per_w)
    n_chunks = len(chunks)
    n_hbm = min(1, n_chunks - 1)
    v_share = (-(-vocab // num_sub) + 7) // 8 * 8
    n_copiers = vocab // v_share
    v_rem = vocab - n_copiers * v_share
    assert v_rem % 8 == 0 and n_copiers + (1 if v_rem else 0) <= num_sub

    mesh = plsc.VectorSubcoreMesh(core_axis_name="c", subcore_axis_name="s")

    @functools.partial(
        pl.kernel,
        mesh=mesh,
        out_type=jax.ShapeDtypeStruct((batch, dim), jnp.float32),
        scratch_types=[
            pltpu.VMEM((b_per_w,), jnp.int32),
            pltpu.VMEM((b_per_w, dim), jnp.float32),
            pltpu.VMEM_SHARED((vocab, dim), jnp.float32),
            pltpu.SemaphoreType.DMA((n_chunks,)),
            pltpu.SemaphoreType.DMA,
            pltpu.SemaphoreType.DMA,
        ],
    )
    def gather_kernel(idx_hbm, table_hbm, out_hbm, idx_v, rows_v, table_sp,
                      gsem, wsem, ssem):
        sid = lax.axis_index("s")
        wid = sid * info.num_cores + lax.axis_index("c")
        base = wid * b_per_w
        if v_rem:
            @pl.when(sid == n_copiers)
            def _copy_rem():
                pltpu.async_copy(
                    table_hbm.at[pl.ds(n_copiers * v_share, v_rem)],
                    table_sp.at[pl.ds(n_copiers * v_share, v_rem)],
                    ssem)

        @pl.when(sid < n_copiers)
        def _copy_share():
            row0 = sid * v_share
            pltpu.async_copy(table_hbm.at[pl.ds(row0, v_share)],
                             table_sp.at[pl.ds(row0, v_share)],
                             ssem)

        pltpu.sync_copy(idx_hbm.at[pl.ds(base, b_per_w)], idx_v)

        gathers = [
            pltpu.async_copy(
                table_hbm.at[idx_v.at[pl.ds(o, s)]],
                rows_v.at[pl.ds(o, s)],
                gsem.at[j],
            )
            for j, (o, s) in enumerate(chunks)
            if j < n_hbm
        ]

        if v_rem:
            @pl.when(sid == n_copiers)
            def _wait_rem():
                pltpu.make_async_copy(
                    table_hbm.at[pl.ds(n_copiers * v_share, v_rem)],
                    table_sp.at[pl.ds(n_copiers * v_share, v_rem)],
                    ssem).wait()

        @pl.when(sid < n_copiers)
        def _wait_share():
            row0 = sid * v_share
            pltpu.make_async_copy(table_hbm.at[pl.ds(row0, v_share)],
                                  table_sp.at[pl.ds(row0, v_share)],
                                  ssem).wait()

        plsc.subcore_barrier()
        gathers += [
            pltpu.async_copy(
                table_sp.at[idx_v.at[pl.ds(o, s)]],
                rows_v.at[pl.ds(o, s)],
                gsem.at[j],
            )
            for j, (o, s) in enumerate(chunks)
            if j >= n_hbm
        ]
        writes = []
        for j, (o, s) in enumerate(chunks):
            gathers[j].wait()
            writes.append(
                pltpu.async_copy(
                    rows_v.at[pl.ds(o, s)],
                    out_hbm.at[pl.ds(base + o, s)],
                    wsem,
                )
            )
        for w in writes:
            w.wait()

    return gather_kernel


def kernel(timesteps, table):
    batch = timesteps.shape[0]
    vocab, dim = table.shape
    return _make_gather(vocab, dim, batch)(timesteps.astype(jnp.int32), table)
